# baseline (speedup 1.0000x reference)
import jax
import jax.numpy as jnp
from jax import lax
from jax.experimental import pallas as pl
from jax.experimental.pallas import tpu as pltpu
from jax.experimental.pallas import tpu_sc as plsc

N_NODES = 10000
N_EDGES = 160000
D = 256
H = 128
NC = 2
NS = 16
CHUNK = 128
E_PAD = 163840
NCHUNKS = E_PAD // CHUNK
CH_PER_SUB = NCHUNKS // NS
CH_PER_WKR = NCHUNKS // (NC * NS)
NBUF = 4
N_PAD = 10240
ROWS_PER_SUB = N_PAD // NS
TRASH = N_NODES
BM = 1000
GRID_M = N_NODES // BM


def _sc_mesh():
    return plsc.VectorSubcoreMesh(core_axis_name="c", subcore_axis_name="s")


def _deg_body(edges3, ones_hbm, zrows, out, ones_v, dst_b, acc,
              sg0, sg1, sg2, sg3):
    c = lax.axis_index("c")
    s = lax.axis_index("s")
    w = c * NS + s
    c0 = w * CH_PER_WKR
    pltpu.sync_copy(ones_hbm, ones_v)
    pltpu.sync_copy(edges3.at[1, pl.ds(c0, CH_PER_WKR), :], dst_b)
    pltpu.sync_copy(zrows, acc.at[pl.ds(s * ROWS_PER_SUB, ROWS_PER_SUB), :])
    plsc.subcore_barrier()
    sems = (sg0, sg1, sg2, sg3)

    def body(t, carry):
        for u in range(NBUF):
            j = t * NBUF + u
            pltpu.async_copy(ones_v, acc.at[dst_b.at[j]], sems[u], add=True)
        return carry

    lax.fori_loop(0, CH_PER_WKR // NBUF, body, 0)

    def drain(t, carry):
        for u in range(NBUF):
            j = t * NBUF + u
            pltpu.make_async_copy(ones_v, acc.at[dst_b.at[j]], sems[u]).wait()
        return carry

    lax.fori_loop(0, CH_PER_WKR // NBUF, drain, 0)
    plsc.subcore_barrier()
    pltpu.sync_copy(acc.at[pl.ds(s * ROWS_PER_SUB, ROWS_PER_SUB), :],
                    out.at[c, pl.ds(s * ROWS_PER_SUB, ROWS_PER_SUB), :])


def _sc_deg(edges3, ones128, z128):
    return pl.kernel(
        _deg_body,
        out_type=jax.ShapeDtypeStruct((NC, N_PAD, H), jnp.float32),
        mesh=_sc_mesh(),
        scratch_types=[
            pltpu.VMEM((CHUNK, H), jnp.float32),
            pltpu.VMEM((CH_PER_WKR, CHUNK), jnp.int32),
            pltpu.VMEM_SHARED((N_PAD, H), jnp.float32),
            pltpu.SemaphoreType.DMA,
            pltpu.SemaphoreType.DMA,
            pltpu.SemaphoreType.DMA,
            pltpu.SemaphoreType.DMA,
        ],
    )(edges3, ones128, z128)


def _scatter_body(g2d, edges_i, zrows, out, i00, i01, i10, i11, r0, r1, acc,
                  si00, si01, si10, si11, sg0, sg1, ss0, ss1):
    c = lax.axis_index("c")
    s = lax.axis_index("s")
    c0 = s * CH_PER_SUB
    idx = ((i00, i01), (i10, i11))
    sem_i = ((si00, si01), (si10, si11))
    rows = (r0, r1)
    sem_g = (sg0, sg1)
    sem_s = (ss0, ss1)
    nt = CH_PER_SUB // 2

    pltpu.sync_copy(zrows, acc.at[pl.ds(s * ROWS_PER_SUB, ROWS_PER_SUB), :])
    plsc.subcore_barrier()

    for u in range(2):
        pltpu.async_copy(edges_i.at[c0 + u], idx[0][u], sem_i[0][u])

    def do_round(t, p, first):
        for u in range(2):
            j = t * 2 + u
            if not first:
                pltpu.make_async_copy(
                    rows[u], acc.at[idx[1 - p][u].at[1]], sem_s[u]).wait()

            @pl.when(t + 1 < nt)
            def _(u=u, j=j):
                pltpu.async_copy(edges_i.at[c0 + j + 2], idx[1 - p][u],
                                 sem_i[1 - p][u])

        for u in range(2):
            j = t * 2 + u
            pltpu.make_async_copy(
                edges_i.at[c0 + j], idx[p][u], sem_i[p][u]).wait()

            @pl.when(c == 1)
            def _(u=u):
                for v in range(CHUNK // 16):
                    sl = pl.ds(v * 16, 16)
                    idx[p][u][0, sl] = idx[p][u][0, sl] + N_PAD

            pltpu.async_copy(g2d.at[idx[p][u].at[0]], rows[u], sem_g[u])

        for u in range(2):
            pltpu.make_async_copy(
                g2d.at[idx[p][u].at[0]], rows[u], sem_g[u]).wait()
            pltpu.async_copy(rows[u], acc.at[idx[p][u].at[1]], sem_s[u],
                             add=True)

    do_round(0, 0, True)

    def body(q, carry):
        t = 1 + q * 2
        do_round(t, 1, False)
        do_round(t + 1, 0, False)
        return carry

    lax.fori_loop(0, (nt - 1) // 2, body, 0)
    do_round(nt - 1, 1, False)

    for u in range(2):
        pltpu.make_async_copy(rows[u], acc.at[idx[1][u].at[1]], sem_s[u]).wait()

    plsc.subcore_barrier()
    pltpu.sync_copy(acc.at[pl.ds(s * ROWS_PER_SUB, ROWS_PER_SUB), :],
                    out.at[c, pl.ds(s * ROWS_PER_SUB, ROWS_PER_SUB), :])


def _sc_scatter(g2d, edges_i, z128):
    return pl.kernel(
        _scatter_body,
        out_type=jax.ShapeDtypeStruct((NC, N_PAD, H), jnp.float32),
        mesh=_sc_mesh(),
        scratch_types=[
            pltpu.VMEM((2, CHUNK), jnp.int32),
            pltpu.VMEM((2, CHUNK), jnp.int32),
            pltpu.VMEM((2, CHUNK), jnp.int32),
            pltpu.VMEM((2, CHUNK), jnp.int32),
            pltpu.VMEM((CHUNK, H), jnp.float32),
            pltpu.VMEM((CHUNK, H), jnp.float32),
            pltpu.VMEM_SHARED((N_PAD, H), jnp.float32),
            pltpu.SemaphoreType.DMA,
            pltpu.SemaphoreType.DMA,
            pltpu.SemaphoreType.DMA,
            pltpu.SemaphoreType.DMA,
            pltpu.SemaphoreType.DMA,
            pltpu.SemaphoreType.DMA,
            pltpu.SemaphoreType.DMA,
            pltpu.SemaphoreType.DMA,
        ],
    )(g2d, edges_i, z128)


def _dinv(dcnt_ref):
    deg = dcnt_ref[0][:, :1] + dcnt_ref[1][:, :1] + 1.0
    return lax.rsqrt(deg)


def _gelu(x):
    return 0.5 * x * (1.0 + lax.erf(x * 0.7071067811865476))


def _first_body(x_ref, w_ref, dcnt_ref, out_ref):
    dinv = _dinv(dcnt_ref)
    h = jnp.dot(x_ref[...], w_ref[...], preferred_element_type=jnp.float32)
    g = dinv * h
    out_ref[0] = g[:, :H]
    out_ref[1] = g[:, H:]


def _mid_body(acc_ref, g_ref, dcnt_ref, b_ref, w_ref, out_ref):
    dinv = _dinv(dcnt_ref)
    pre = jnp.concatenate([acc_ref[0] + g_ref[0], acc_ref[1] + g_ref[1]], axis=1)
    a = _gelu(dinv * pre + b_ref[...])
    hn = jnp.dot(a, w_ref[...], preferred_element_type=jnp.float32)
    gn = dinv * hn
    out_ref[0] = gn[:, :H]
    out_ref[1] = gn[:, H:]


def _final_body(acc_ref, g_ref, dcnt_ref, b_ref, out_ref):
    dinv = _dinv(dcnt_ref)
    pre = jnp.concatenate([acc_ref[0] + g_ref[0], acc_ref[1] + g_ref[1]], axis=1)
    out_ref[...] = dinv * pre + b_ref[...]


_DCNT_SPEC = pl.BlockSpec((NC, BM, H), lambda i: (0, i, 0))
_HALVES_SPEC = pl.BlockSpec((NC, BM, H), lambda i: (0, i, 0))
_W_SPEC = pl.BlockSpec((D, D), lambda i: (0, 0))
_B_SPEC = pl.BlockSpec((1, D), lambda i: (0, 0))


def _tc_first(x, W1, dcnt):
    return pl.pallas_call(
        _first_body,
        grid=(GRID_M,),
        in_specs=[pl.BlockSpec((BM, D), lambda i: (i, 0)), _W_SPEC, _DCNT_SPEC],
        out_specs=_HALVES_SPEC,
        out_shape=jax.ShapeDtypeStruct((NC, N_PAD, H), jnp.float32),
    )(x, W1, dcnt)


def _tc_mid(acc, g, dcnt, b, W):
    return pl.pallas_call(
        _mid_body,
        grid=(GRID_M,),
        in_specs=[_HALVES_SPEC, _HALVES_SPEC, _DCNT_SPEC, _B_SPEC, _W_SPEC],
        out_specs=_HALVES_SPEC,
        out_shape=jax.ShapeDtypeStruct((NC, N_PAD, H), jnp.float32),
    )(acc, g, dcnt, b, W)


def _tc_final(acc, g, dcnt, b):
    return pl.pallas_call(
        _final_body,
        grid=(GRID_M,),
        in_specs=[_HALVES_SPEC, _HALVES_SPEC, _DCNT_SPEC, _B_SPEC],
        out_specs=pl.BlockSpec((BM, D), lambda i: (i, 0)),
        out_shape=jax.ShapeDtypeStruct((N_NODES, D), jnp.float32),
    )(acc, g, dcnt, b)


def kernel(x, edge_index, W1, b1, W2, b2, W3, b3):
    ones128 = jnp.ones((CHUNK, H), jnp.float32)
    z128 = jnp.zeros((ROWS_PER_SUB, H), jnp.float32)
    pad = jnp.stack([
        jnp.zeros((E_PAD - N_EDGES,), jnp.int32),
        jnp.full((E_PAD - N_EDGES,), TRASH, jnp.int32),
    ])
    edges3 = jnp.concatenate([edge_index, pad], axis=1).reshape(2, NCHUNKS, CHUNK)
    edges_i = edges3.transpose(1, 0, 2)
    dcnt = _sc_deg(edges3, ones128, z128)
    g1 = _tc_first(x, W1, dcnt)
    acc1 = _sc_scatter(g1.reshape(NC * N_PAD, H), edges_i, z128)
    g2 = _tc_mid(acc1, g1, dcnt, b1.reshape(1, D), W2)
    acc2 = _sc_scatter(g2.reshape(NC * N_PAD, H), edges_i, z128)
    g3 = _tc_mid(acc2, g2, dcnt, b2.reshape(1, D), W3)
    acc3 = _sc_scatter(g3.reshape(NC * N_PAD, H), edges_i, z128)
    return _tc_final(acc3, g3, dcnt, b3.reshape(1, D))


# speedup vs baseline: 1.7943x; 1.7943x over previous
import jax
import jax.numpy as jnp
from jax import lax
from jax.experimental import pallas as pl
from jax.experimental.pallas import tpu as pltpu
from jax.experimental.pallas import tpu_sc as plsc

N_NODES = 10000
N_EDGES = 160000
D = 256
H = 128
NC = 2
NS = 16
CHUNK = 128
E_PAD = 163840
NCHUNKS = E_PAD // CHUNK
CH_PER_SUB = NCHUNKS // NS
CH_PER_WKR = NCHUNKS // (NC * NS)
NBUF = 4
N_PAD = 10240
ROWS_PER_SUB = N_PAD // NS
TRASH = N_NODES
BM = 1000
GRID_M = N_NODES // BM


def _sc_mesh():
    return plsc.VectorSubcoreMesh(core_axis_name="c", subcore_axis_name="s")


def _deg_body(edges3, ones_hbm, zrows, out, ones_v, dst_b, acc,
              sg0, sg1, sg2, sg3):
    c = lax.axis_index("c")
    s = lax.axis_index("s")
    w = c * NS + s
    c0 = w * CH_PER_WKR
    pltpu.sync_copy(ones_hbm, ones_v)
    pltpu.sync_copy(edges3.at[1, pl.ds(c0, CH_PER_WKR), :], dst_b)
    pltpu.sync_copy(zrows, acc.at[pl.ds(s * ROWS_PER_SUB, ROWS_PER_SUB), :])
    plsc.subcore_barrier()
    sems = (sg0, sg1, sg2, sg3)

    def body(t, carry):
        for u in range(NBUF):
            j = t * NBUF + u
            pltpu.async_copy(ones_v, acc.at[dst_b.at[j]], sems[u], add=True)
        return carry

    lax.fori_loop(0, CH_PER_WKR // NBUF, body, 0)

    def drain(t, carry):
        for u in range(NBUF):
            j = t * NBUF + u
            pltpu.make_async_copy(ones_v, acc.at[dst_b.at[j]], sems[u]).wait()
        return carry

    lax.fori_loop(0, CH_PER_WKR // NBUF, drain, 0)
    plsc.subcore_barrier()
    pltpu.sync_copy(acc.at[pl.ds(s * ROWS_PER_SUB, ROWS_PER_SUB), :],
                    out.at[c, pl.ds(s * ROWS_PER_SUB, ROWS_PER_SUB), :])


def _sc_deg(edges3, ones128, z128):
    return pl.kernel(
        _deg_body,
        out_type=jax.ShapeDtypeStruct((NC, N_PAD, H), jnp.float32),
        mesh=_sc_mesh(),
        scratch_types=[
            pltpu.VMEM((CHUNK, H), jnp.float32),
            pltpu.VMEM((CH_PER_WKR, CHUNK), jnp.int32),
            pltpu.VMEM_SHARED((N_PAD, H), jnp.float32),
            pltpu.SemaphoreType.DMA,
            pltpu.SemaphoreType.DMA,
            pltpu.SemaphoreType.DMA,
            pltpu.SemaphoreType.DMA,
        ],
    )(edges3, ones128, z128)


def _scatter_body(g2d, edges_i, zrows, out, i00, i01, r0, r1, acc,
                  sg0, sg1, ss0, ss1):
    c = lax.axis_index("c")
    s = lax.axis_index("s")
    idx = (i00, i01)
    rows = (r0, r1)
    sem_g = (sg0, sg1)
    sem_s = (ss0, ss1)

    pltpu.sync_copy(zrows, acc.at[pl.ds(s * ROWS_PER_SUB, ROWS_PER_SUB), :])
    plsc.subcore_barrier()

    def chunk(k, p, q):
        @pl.when(q > 0)
        def _():
            pltpu.make_async_copy(rows[p], acc.at[idx[p].at[1]],
                                  sem_s[p]).wait()

        pltpu.sync_copy(edges_i.at[k], idx[p])

        @pl.when(c == 1)
        def _():
            for v in range(CHUNK // 16):
                sl = pl.ds(v * 16, 16)
                idx[p][0, sl] = idx[p][0, sl] + N_PAD

        pltpu.async_copy(g2d.at[idx[p].at[0]], rows[p], sem_g[p]).wait()
        pltpu.async_copy(rows[p], acc.at[idx[p].at[1]], sem_s[p], add=True)

    def body(q, carry):
        chunk(s + NS * (2 * q), 0, q)
        chunk(s + NS * (2 * q + 1), 1, q)
        return carry

    lax.fori_loop(0, CH_PER_SUB // 2, body, 0)
    for p in range(2):
        pltpu.make_async_copy(rows[p], acc.at[idx[p].at[1]], sem_s[p]).wait()

    plsc.subcore_barrier()
    pltpu.sync_copy(acc.at[pl.ds(s * ROWS_PER_SUB, ROWS_PER_SUB), :],
                    out.at[c, pl.ds(s * ROWS_PER_SUB, ROWS_PER_SUB), :])


def _sc_scatter(g2d, edges_i, z128):
    return pl.kernel(
        _scatter_body,
        out_type=jax.ShapeDtypeStruct((NC, N_PAD, H), jnp.float32),
        mesh=_sc_mesh(),
        scratch_types=[
            pltpu.VMEM((2, CHUNK), jnp.int32),
            pltpu.VMEM((2, CHUNK), jnp.int32),
            pltpu.VMEM((CHUNK, H), jnp.float32),
            pltpu.VMEM((CHUNK, H), jnp.float32),
            pltpu.VMEM_SHARED((N_PAD, H), jnp.float32),
            pltpu.SemaphoreType.DMA,
            pltpu.SemaphoreType.DMA,
            pltpu.SemaphoreType.DMA,
            pltpu.SemaphoreType.DMA,
        ],
    )(g2d, edges_i, z128)


def _dinv(dcnt_ref):
    deg = dcnt_ref[0][:, :1] + dcnt_ref[1][:, :1] + 1.0
    return lax.rsqrt(deg)


def _gelu(x):
    return 0.5 * x * (1.0 + lax.erf(x * 0.7071067811865476))


def _first_body(x_ref, w_ref, dcnt_ref, out_ref):
    dinv = _dinv(dcnt_ref)
    h = jnp.dot(x_ref[...], w_ref[...], preferred_element_type=jnp.float32)
    g = dinv * h
    out_ref[0] = g[:, :H]
    out_ref[1] = g[:, H:]


def _mid_body(acc_ref, g_ref, dcnt_ref, b_ref, w_ref, out_ref):
    dinv = _dinv(dcnt_ref)
    pre = jnp.concatenate([acc_ref[0] + g_ref[0], acc_ref[1] + g_ref[1]], axis=1)
    a = _gelu(dinv * pre + b_ref[...])
    hn = jnp.dot(a, w_ref[...], preferred_element_type=jnp.float32)
    gn = dinv * hn
    out_ref[0] = gn[:, :H]
    out_ref[1] = gn[:, H:]


def _final_body(acc_ref, g_ref, dcnt_ref, b_ref, out_ref):
    dinv = _dinv(dcnt_ref)
    pre = jnp.concatenate([acc_ref[0] + g_ref[0], acc_ref[1] + g_ref[1]], axis=1)
    out_ref[...] = dinv * pre + b_ref[...]


_DCNT_SPEC = pl.BlockSpec((NC, BM, H), lambda i: (0, i, 0))
_HALVES_SPEC = pl.BlockSpec((NC, BM, H), lambda i: (0, i, 0))
_W_SPEC = pl.BlockSpec((D, D), lambda i: (0, 0))
_B_SPEC = pl.BlockSpec((1, D), lambda i: (0, 0))


def _tc_first(x, W1, dcnt):
    return pl.pallas_call(
        _first_body,
        grid=(GRID_M,),
        in_specs=[pl.BlockSpec((BM, D), lambda i: (i, 0)), _W_SPEC, _DCNT_SPEC],
        out_specs=_HALVES_SPEC,
        out_shape=jax.ShapeDtypeStruct((NC, N_PAD, H), jnp.float32),
    )(x, W1, dcnt)


def _tc_mid(acc, g, dcnt, b, W):
    return pl.pallas_call(
        _mid_body,
        grid=(GRID_M,),
        in_specs=[_HALVES_SPEC, _HALVES_SPEC, _DCNT_SPEC, _B_SPEC, _W_SPEC],
        out_specs=_HALVES_SPEC,
        out_shape=jax.ShapeDtypeStruct((NC, N_PAD, H), jnp.float32),
    )(acc, g, dcnt, b, W)


def _tc_final(acc, g, dcnt, b):
    return pl.pallas_call(
        _final_body,
        grid=(GRID_M,),
        in_specs=[_HALVES_SPEC, _HALVES_SPEC, _DCNT_SPEC, _B_SPEC],
        out_specs=pl.BlockSpec((BM, D), lambda i: (i, 0)),
        out_shape=jax.ShapeDtypeStruct((N_NODES, D), jnp.float32),
    )(acc, g, dcnt, b)


def kernel(x, edge_index, W1, b1, W2, b2, W3, b3):
    ones128 = jnp.ones((CHUNK, H), jnp.float32)
    z128 = jnp.zeros((ROWS_PER_SUB, H), jnp.float32)
    npad = E_PAD - N_EDGES
    spread = jnp.arange(npad, dtype=jnp.int32)
    pad = jnp.stack([
        spread % N_NODES,
        TRASH + spread % (N_PAD - N_NODES),
    ])
    edges3 = jnp.concatenate([edge_index, pad], axis=1).reshape(2, NCHUNKS, CHUNK)
    edges_i = edges3.transpose(1, 0, 2)
    dcnt = _sc_deg(edges3, ones128, z128)
    g1 = _tc_first(x, W1, dcnt)
    acc1 = _sc_scatter(g1.reshape(NC * N_PAD, H), edges_i, z128)
    g2 = _tc_mid(acc1, g1, dcnt, b1.reshape(1, D), W2)
    acc2 = _sc_scatter(g2.reshape(NC * N_PAD, H), edges_i, z128)
    g3 = _tc_mid(acc2, g2, dcnt, b2.reshape(1, D), W3)
    acc3 = _sc_scatter(g3.reshape(NC * N_PAD, H), edges_i, z128)
    return _tc_final(acc3, g3, dcnt, b3.reshape(1, D))


# speedup vs baseline: 2.1167x; 1.1797x over previous
import jax
import jax.numpy as jnp
from jax import lax
from jax.experimental import pallas as pl
from jax.experimental.pallas import tpu as pltpu
from jax.experimental.pallas import tpu_sc as plsc

N_NODES = 10000
N_EDGES = 160000
D = 256
H = 128
NC = 2
NS = 16
CHUNK = 128
E_PAD = 163840
NCHUNKS = E_PAD // CHUNK
CH_PER_SUB = NCHUNKS // NS
CH_PER_WKR = NCHUNKS // (NC * NS)
NBUF = 4
N_PAD = 10240
ROWS_PER_SUB = N_PAD // NS
TRASH = N_NODES
BM = 1000
GRID_M = N_NODES // BM


def _sc_mesh():
    return plsc.VectorSubcoreMesh(core_axis_name="c", subcore_axis_name="s")


def _deg_body(edges3, ones_hbm, zrows, out, ones_v, dst_b, acc,
              sg0, sg1, sg2, sg3):
    c = lax.axis_index("c")
    s = lax.axis_index("s")
    w = c * NS + s
    c0 = w * CH_PER_WKR
    pltpu.sync_copy(ones_hbm, ones_v)
    pltpu.sync_copy(edges3.at[1, pl.ds(c0, CH_PER_WKR), :], dst_b)
    pltpu.sync_copy(zrows, acc.at[pl.ds(s * ROWS_PER_SUB, ROWS_PER_SUB), :])
    plsc.subcore_barrier()
    sems = (sg0, sg1, sg2, sg3)

    def body(t, carry):
        for u in range(NBUF):
            j = t * NBUF + u
            pltpu.async_copy(ones_v, acc.at[dst_b.at[j]], sems[u], add=True)
        return carry

    lax.fori_loop(0, CH_PER_WKR // NBUF, body, 0)

    def drain(t, carry):
        for u in range(NBUF):
            j = t * NBUF + u
            pltpu.make_async_copy(ones_v, acc.at[dst_b.at[j]], sems[u]).wait()
        return carry

    lax.fori_loop(0, CH_PER_WKR // NBUF, drain, 0)
    plsc.subcore_barrier()
    pltpu.sync_copy(acc.at[pl.ds(s * ROWS_PER_SUB, ROWS_PER_SUB), :],
                    out.at[c, pl.ds(s * ROWS_PER_SUB, ROWS_PER_SUB), :])


def _sc_deg(edges3, ones128, z128):
    return pl.kernel(
        _deg_body,
        out_type=jax.ShapeDtypeStruct((NC, N_PAD, H), jnp.float32),
        mesh=_sc_mesh(),
        scratch_types=[
            pltpu.VMEM((CHUNK, H), jnp.float32),
            pltpu.VMEM((CH_PER_WKR, CHUNK), jnp.int32),
            pltpu.VMEM_SHARED((N_PAD, H), jnp.float32),
            pltpu.SemaphoreType.DMA,
            pltpu.SemaphoreType.DMA,
            pltpu.SemaphoreType.DMA,
            pltpu.SemaphoreType.DMA,
        ],
    )(edges3, ones128, z128)


def _scatter_body(g2d, edges_i, zrows, out, i00, i01, r0, r1, acc,
                  sg0, sg1, ss0, ss1):
    c = lax.axis_index("c")
    s = lax.axis_index("s")
    idx = (i00, i01)
    rows = (r0, r1)
    sem_g = (sg0, sg1)
    sem_s = (ss0, ss1)

    pltpu.sync_copy(zrows, acc.at[pl.ds(s * ROWS_PER_SUB, ROWS_PER_SUB), :])
    plsc.subcore_barrier()

    def stage(k, p, q):
        @pl.when(q > 0)
        def _():
            pltpu.make_async_copy(rows[p], acc.at[idx[p].at[1]],
                                  sem_s[p]).wait()

        pltpu.sync_copy(edges_i.at[k], idx[p])

        @pl.when(c == 1)
        def _():
            for v in range(CHUNK // 16):
                sl = pl.ds(v * 16, 16)
                idx[p][0, sl] = idx[p][0, sl] + N_PAD

        pltpu.async_copy(g2d.at[idx[p].at[0]], rows[p], sem_g[p])

    def drain(p):
        pltpu.make_async_copy(g2d.at[idx[p].at[0]], rows[p], sem_g[p]).wait()
        pltpu.async_copy(rows[p], acc.at[idx[p].at[1]], sem_s[p], add=True)

    def body(q, carry):
        stage(s + NS * (2 * q), 0, q)
        stage(s + NS * (2 * q + 1), 1, q)
        drain(0)
        drain(1)
        return carry

    lax.fori_loop(0, CH_PER_SUB // 2, body, 0)
    for p in range(2):
        pltpu.make_async_copy(rows[p], acc.at[idx[p].at[1]], sem_s[p]).wait()

    plsc.subcore_barrier()
    pltpu.sync_copy(acc.at[pl.ds(s * ROWS_PER_SUB, ROWS_PER_SUB), :],
                    out.at[c, pl.ds(s * ROWS_PER_SUB, ROWS_PER_SUB), :])


def _sc_scatter(g2d, edges_i, z128):
    return pl.kernel(
        _scatter_body,
        out_type=jax.ShapeDtypeStruct((NC, N_PAD, H), jnp.float32),
        mesh=_sc_mesh(),
        scratch_types=[
            pltpu.VMEM((2, CHUNK), jnp.int32),
            pltpu.VMEM((2, CHUNK), jnp.int32),
            pltpu.VMEM((CHUNK, H), jnp.float32),
            pltpu.VMEM((CHUNK, H), jnp.float32),
            pltpu.VMEM_SHARED((N_PAD, H), jnp.float32),
            pltpu.SemaphoreType.DMA,
            pltpu.SemaphoreType.DMA,
            pltpu.SemaphoreType.DMA,
            pltpu.SemaphoreType.DMA,
        ],
    )(g2d, edges_i, z128)


def _dinv(dcnt_ref):
    deg = dcnt_ref[0][:, :1] + dcnt_ref[1][:, :1] + 1.0
    return lax.rsqrt(deg)


def _gelu(x):
    return 0.5 * x * (1.0 + lax.erf(x * 0.7071067811865476))


def _first_body(x_ref, w_ref, dcnt_ref, out_ref):
    dinv = _dinv(dcnt_ref)
    h = jnp.dot(x_ref[...], w_ref[...], preferred_element_type=jnp.float32)
    g = dinv * h
    out_ref[0] = g[:, :H]
    out_ref[1] = g[:, H:]


def _mid_body(acc_ref, g_ref, dcnt_ref, b_ref, w_ref, out_ref):
    dinv = _dinv(dcnt_ref)
    pre = jnp.concatenate([acc_ref[0] + g_ref[0], acc_ref[1] + g_ref[1]], axis=1)
    a = _gelu(dinv * pre + b_ref[...])
    hn = jnp.dot(a, w_ref[...], preferred_element_type=jnp.float32)
    gn = dinv * hn
    out_ref[0] = gn[:, :H]
    out_ref[1] = gn[:, H:]


def _final_body(acc_ref, g_ref, dcnt_ref, b_ref, out_ref):
    dinv = _dinv(dcnt_ref)
    pre = jnp.concatenate([acc_ref[0] + g_ref[0], acc_ref[1] + g_ref[1]], axis=1)
    out_ref[...] = dinv * pre + b_ref[...]


_DCNT_SPEC = pl.BlockSpec((NC, BM, H), lambda i: (0, i, 0))
_HALVES_SPEC = pl.BlockSpec((NC, BM, H), lambda i: (0, i, 0))
_W_SPEC = pl.BlockSpec((D, D), lambda i: (0, 0))
_B_SPEC = pl.BlockSpec((1, D), lambda i: (0, 0))


def _tc_first(x, W1, dcnt):
    return pl.pallas_call(
        _first_body,
        grid=(GRID_M,),
        in_specs=[pl.BlockSpec((BM, D), lambda i: (i, 0)), _W_SPEC, _DCNT_SPEC],
        out_specs=_HALVES_SPEC,
        out_shape=jax.ShapeDtypeStruct((NC, N_PAD, H), jnp.float32),
    )(x, W1, dcnt)


def _tc_mid(acc, g, dcnt, b, W):
    return pl.pallas_call(
        _mid_body,
        grid=(GRID_M,),
        in_specs=[_HALVES_SPEC, _HALVES_SPEC, _DCNT_SPEC, _B_SPEC, _W_SPEC],
        out_specs=_HALVES_SPEC,
        out_shape=jax.ShapeDtypeStruct((NC, N_PAD, H), jnp.float32),
    )(acc, g, dcnt, b, W)


def _tc_final(acc, g, dcnt, b):
    return pl.pallas_call(
        _final_body,
        grid=(GRID_M,),
        in_specs=[_HALVES_SPEC, _HALVES_SPEC, _DCNT_SPEC, _B_SPEC],
        out_specs=pl.BlockSpec((BM, D), lambda i: (i, 0)),
        out_shape=jax.ShapeDtypeStruct((N_NODES, D), jnp.float32),
    )(acc, g, dcnt, b)


def kernel(x, edge_index, W1, b1, W2, b2, W3, b3):
    ones128 = jnp.ones((CHUNK, H), jnp.float32)
    z128 = jnp.zeros((ROWS_PER_SUB, H), jnp.float32)
    npad = E_PAD - N_EDGES
    spread = jnp.arange(npad, dtype=jnp.int32)
    pad = jnp.stack([
        spread % N_NODES,
        TRASH + spread % (N_PAD - N_NODES),
    ])
    edges3 = jnp.concatenate([edge_index, pad], axis=1).reshape(2, NCHUNKS, CHUNK)
    edges_i = edges3.transpose(1, 0, 2)
    dcnt = _sc_deg(edges3, ones128, z128)
    g1 = _tc_first(x, W1, dcnt)
    acc1 = _sc_scatter(g1.reshape(NC * N_PAD, H), edges_i, z128)
    g2 = _tc_mid(acc1, g1, dcnt, b1.reshape(1, D), W2)
    acc2 = _sc_scatter(g2.reshape(NC * N_PAD, H), edges_i, z128)
    g3 = _tc_mid(acc2, g2, dcnt, b2.reshape(1, D), W3)
    acc3 = _sc_scatter(g3.reshape(NC * N_PAD, H), edges_i, z128)
    return _tc_final(acc3, g3, dcnt, b3.reshape(1, D))
